# baseline (speedup 1.0000x reference)
import functools

import jax
import jax.numpy as jnp
from jax import lax
from jax.experimental import pallas as pl
from jax.experimental.pallas import tpu as pltpu
from jax.experimental.pallas import tpu_sc as plsc

EMB = 64
HID = 128
P = 3
B = 4096


def _sc_gather3(user_tab, item_tab, users, pos, neg):
    info = plsc.get_sparse_core_info()
    _NC, _NS = info.num_cores, info.num_subcores
    _NW = _NC * _NS
    _BPW = B // _NW
    mesh = plsc.VectorSubcoreMesh(core_axis_name="c", subcore_axis_name="s")

    @functools.partial(
        pl.kernel,
        mesh=mesh,
        out_type=[jax.ShapeDtypeStruct((B, HID), jnp.float32)] * (3 * P),
        scratch_types=[
            pltpu.VMEM((_BPW,), jnp.int32),
            pltpu.VMEM((_BPW, P, HID), jnp.float32),
            pltpu.SemaphoreType.DMA,
        ],
        compiler_params=pltpu.CompilerParams(use_tc_tiling_on_sc=True),
    )
    def gather3(utab, itab, u_idx, p_idx, n_idx, *outs_scratch):
        outs = outs_scratch[:3 * P]
        idx_v, rows_v, sem = outs_scratch[3 * P:]
        wid = lax.axis_index("s") * _NC + lax.axis_index("c")
        base = wid * _BPW
        for j, (idx_hbm, tab) in enumerate(((u_idx, utab),
                                            (p_idx, itab),
                                            (n_idx, itab))):
            pltpu.sync_copy(idx_hbm.at[pl.ds(base, _BPW)], idx_v)
            pltpu.async_copy(tab.at[idx_v], rows_v, sem).wait()
            for k in range(P):
                pltpu.sync_copy(rows_v.at[:, k],
                                outs[3 * j + k].at[pl.ds(base, _BPW)])

    return gather3(user_tab, item_tab, users, pos, neg)


def _max3(a, b, c):
    return jnp.maximum(jnp.maximum(a, b), c)


def _dense_body(u0_ref, u1_ref, u2_ref, p0_ref, p1_ref, p2_ref,
                n0_ref, n1_ref, n2_ref, wu_ref, wi_ref, a_ref,
                pu_ref, pi_ref, nu_ref, ni_ref):
    wu = wu_ref[...]
    wi = wi_ref[...]
    a = a_ref[...]
    proj_u = [jnp.dot(r[...], wu) for r in (u0_ref, u1_ref, u2_ref)]
    m_tmp = [jnp.dot(x, a) for x in proj_u]
    for i_refs, uo_ref, io_ref in (((p0_ref, p1_ref, p2_ref), pu_ref, pi_ref),
                                   ((n0_ref, n1_ref, n2_ref), nu_ref, ni_ref)):
        proj_i = [jnp.dot(r[...], wi) for r in i_refs]
        m = [[jnp.sum(m_tmp[p] * proj_i[q], axis=1, keepdims=True)
              for q in range(P)] for p in range(P)]
        u_logit = [_max3(m[p][0], m[p][1], m[p][2]) for p in range(P)]
        i_logit = [_max3(m[0][q], m[1][q], m[2][q]) for q in range(P)]
        um = _max3(*u_logit)
        ue = [jnp.exp(x - um) for x in u_logit]
        us = ue[0] + ue[1] + ue[2]
        uo_ref[...] = (ue[0] * proj_u[0] + ue[1] * proj_u[1]
                       + ue[2] * proj_u[2]) / us
        im = _max3(*i_logit)
        ie = [jnp.exp(x - im) for x in i_logit]
        isum = ie[0] + ie[1] + ie[2]
        io_ref[...] = (ie[0] * proj_i[0] + ie[1] * proj_i[1]
                       + ie[2] * proj_i[2]) / isum


def _dense_coattention(rows, W_u, W_i, A):
    BT = 512
    row_spec = pl.BlockSpec((BT, HID), lambda i: (i, 0))
    full = lambda shape: pl.BlockSpec(shape, lambda i: (0, 0))
    return pl.pallas_call(
        _dense_body,
        grid=(B // BT,),
        in_specs=[row_spec] * 9 + [full((HID, EMB)), full((HID, EMB)),
                                   full((EMB, EMB))],
        out_specs=[pl.BlockSpec((BT, EMB), lambda i: (i, 0))] * 4,
        out_shape=[jax.ShapeDtypeStruct((B, EMB), jnp.float32)] * 4,
    )(*rows, W_u, W_i, A)


def kernel(users, pos_items, neg_items, multi_user_embed, multi_item_embed,
           W_u, W_i, A):
    rows = _sc_gather3(
        multi_user_embed, multi_item_embed,
        users.astype(jnp.int32), pos_items.astype(jnp.int32),
        neg_items.astype(jnp.int32))
    pu, pi, nu, ni = _dense_coattention(rows, W_u, W_i, A)
    return (pu, pi, nu, ni)


# speedup vs baseline: 8.9900x; 8.9900x over previous
import functools

import jax
import jax.numpy as jnp
from jax import lax
from jax.experimental import pallas as pl
from jax.experimental.pallas import tpu as pltpu
from jax.experimental.pallas import tpu_sc as plsc

EMB = 64
HID = 128
P = 3
B = 4096


def _sc_gather3(user_tab, item_tab, users, pos, neg):
    info = plsc.get_sparse_core_info()
    _NC, _NS = info.num_cores, info.num_subcores
    _NW = _NC * _NS
    _BPW = B // _NW
    mesh = plsc.VectorSubcoreMesh(core_axis_name="c", subcore_axis_name="s")

    @functools.partial(
        pl.kernel,
        mesh=mesh,
        out_type=[jax.ShapeDtypeStruct((B, HID), jnp.float32)] * (3 * P),
        scratch_types=[
            pltpu.VMEM((_BPW,), jnp.int32),
            pltpu.VMEM((_BPW, HID), jnp.float32),
            pltpu.SemaphoreType.DMA,
        ],
    )
    def gather3(utab, itab, u_idx, p_idx, n_idx, *outs_scratch):
        outs = outs_scratch[:3 * P]
        idx_v, row_v, sem = outs_scratch[3 * P:]
        wid = lax.axis_index("s") * _NC + lax.axis_index("c")
        base = wid * _BPW
        for j, (idx_hbm, tab) in enumerate(((u_idx, utab),
                                            (p_idx, itab),
                                            (n_idx, itab))):
            pltpu.sync_copy(idx_hbm.at[pl.ds(base, _BPW)], idx_v)
            for k in range(P):
                pltpu.async_copy(tab.at[k].at[idx_v], row_v, sem).wait()
                pltpu.sync_copy(row_v, outs[3 * j + k].at[pl.ds(base, _BPW)])

    return gather3(user_tab, item_tab, users, pos, neg)


def _max3(a, b, c):
    return jnp.maximum(jnp.maximum(a, b), c)


def _dense_body(u0_ref, u1_ref, u2_ref, p0_ref, p1_ref, p2_ref,
                n0_ref, n1_ref, n2_ref, wu_ref, wi_ref, a_ref,
                pu_ref, pi_ref, nu_ref, ni_ref):
    wu = wu_ref[...]
    wi = wi_ref[...]
    a = a_ref[...]
    proj_u = [jnp.dot(r[...], wu) for r in (u0_ref, u1_ref, u2_ref)]
    m_tmp = [jnp.dot(x, a) for x in proj_u]
    for i_refs, uo_ref, io_ref in (((p0_ref, p1_ref, p2_ref), pu_ref, pi_ref),
                                   ((n0_ref, n1_ref, n2_ref), nu_ref, ni_ref)):
        proj_i = [jnp.dot(r[...], wi) for r in i_refs]
        m = [[jnp.sum(m_tmp[p] * proj_i[q], axis=1, keepdims=True)
              for q in range(P)] for p in range(P)]
        u_logit = [_max3(m[p][0], m[p][1], m[p][2]) for p in range(P)]
        i_logit = [_max3(m[0][q], m[1][q], m[2][q]) for q in range(P)]
        um = _max3(*u_logit)
        ue = [jnp.exp(x - um) for x in u_logit]
        us = ue[0] + ue[1] + ue[2]
        uo_ref[...] = (ue[0] * proj_u[0] + ue[1] * proj_u[1]
                       + ue[2] * proj_u[2]) / us
        im = _max3(*i_logit)
        ie = [jnp.exp(x - im) for x in i_logit]
        isum = ie[0] + ie[1] + ie[2]
        io_ref[...] = (ie[0] * proj_i[0] + ie[1] * proj_i[1]
                       + ie[2] * proj_i[2]) / isum


def _dense_coattention(rows, W_u, W_i, A):
    BT = 512
    row_spec = pl.BlockSpec((BT, HID), lambda i: (i, 0))
    full = lambda shape: pl.BlockSpec(shape, lambda i: (0, 0))
    return pl.pallas_call(
        _dense_body,
        grid=(B // BT,),
        in_specs=[row_spec] * 9 + [full((HID, EMB)), full((HID, EMB)),
                                   full((EMB, EMB))],
        out_specs=[pl.BlockSpec((BT, EMB), lambda i: (i, 0))] * 4,
        out_shape=[jax.ShapeDtypeStruct((B, EMB), jnp.float32)] * 4,
    )(*rows, W_u, W_i, A)


def kernel(users, pos_items, neg_items, multi_user_embed, multi_item_embed,
           W_u, W_i, A):
    t_utab = jnp.transpose(multi_user_embed, (1, 0, 2))
    t_itab = jnp.transpose(multi_item_embed, (1, 0, 2))
    rows = _sc_gather3(
        t_utab, t_itab,
        users.astype(jnp.int32), pos_items.astype(jnp.int32),
        neg_items.astype(jnp.int32))
    pu, pi, nu, ni = _dense_coattention(rows, W_u, W_i, A)
    return (pu, pi, nu, ni)
